# baseline (speedup 1.0000x reference)
import functools

import jax
import jax.numpy as jnp
from jax import lax
from jax.experimental import pallas as pl
from jax.experimental.pallas import tpu as pltpu
from jax.experimental.pallas import tpu_sc as plsc

NUM_RELS = 100000
EMB_DIM = 512
BATCH = 16384

_info = plsc.get_sparse_core_info()
_NC, _NS = _info.num_cores, _info.num_subcores
NW = _NC * _NS
B_PER_W = BATCH // NW
CHUNK = 64
NCHUNK = B_PER_W // CHUNK

_mesh = plsc.VectorSubcoreMesh(core_axis_name="c", subcore_axis_name="s")


@functools.partial(
    pl.kernel,
    mesh=_mesh,
    out_type=jax.ShapeDtypeStruct((BATCH, EMB_DIM), jnp.float32),
    scratch_types=[
        pltpu.VMEM((NCHUNK, CHUNK), jnp.int32),
        pltpu.VMEM((2, CHUNK, EMB_DIM), jnp.float32),
        pltpu.SemaphoreType.DMA,
        pltpu.SemaphoreType.DMA,
        pltpu.SemaphoreType.DMA,
        pltpu.SemaphoreType.DMA,
    ],
)
def _gather_kernel(idx_hbm, table_hbm, out_hbm, idx_v, rows_v, g0, g1, w0, w1):
    wid = lax.axis_index("s") * _NC + lax.axis_index("c")
    base = wid * B_PER_W
    for j in range(NCHUNK):
        pltpu.sync_copy(idx_hbm.at[pl.ds(base + j * CHUNK, CHUNK)], idx_v.at[j])
    gsem = (g0, g1)
    wsem = (w0, w1)
    gh = [None, None]
    wh = [None, None]
    gh[0] = pltpu.async_copy(table_hbm.at[idx_v.at[0]], rows_v.at[0], gsem[0])
    for j in range(NCHUNK):
        b = j % 2
        nb = 1 - b
        if j + 1 < NCHUNK:
            if wh[nb] is not None:
                wh[nb].wait()
            gh[nb] = pltpu.async_copy(
                table_hbm.at[idx_v.at[j + 1]], rows_v.at[nb], gsem[nb])
        gh[b].wait()
        wh[b] = pltpu.async_copy(
            rows_v.at[b], out_hbm.at[pl.ds(base + j * CHUNK, CHUNK)], wsem[b])
    for h in wh:
        if h is not None:
            h.wait()


def kernel(rel_ids, clip_embs):
    return _gather_kernel(rel_ids.astype(jnp.int32), clip_embs)


# speedup vs baseline: 1.0634x; 1.0634x over previous
import functools

import jax
import jax.numpy as jnp
from jax import lax
from jax.experimental import pallas as pl
from jax.experimental.pallas import tpu as pltpu
from jax.experimental.pallas import tpu_sc as plsc

NUM_RELS = 100000
EMB_DIM = 512
BATCH = 16384

_info = plsc.get_sparse_core_info()
_NC, _NS = _info.num_cores, _info.num_subcores
NW = _NC * _NS
B_PER_W = BATCH // NW
CHUNK = 64
NCHUNK = B_PER_W // CHUNK

_mesh = plsc.VectorSubcoreMesh(core_axis_name="c", subcore_axis_name="s")


@functools.partial(
    pl.kernel,
    mesh=_mesh,
    out_type=jax.ShapeDtypeStruct((BATCH, EMB_DIM), jnp.float32),
    scratch_types=[
        pltpu.VMEM((NCHUNK, CHUNK), jnp.int32),
        pltpu.VMEM((2, CHUNK, EMB_DIM), jnp.float32),
        pltpu.SemaphoreType.DMA,
        pltpu.SemaphoreType.DMA,
        pltpu.SemaphoreType.DMA,
        pltpu.SemaphoreType.DMA,
        pltpu.SemaphoreType.DMA,
    ],
)
def _gather_kernel(idx_hbm, table_hbm, out_hbm, idx_v, rows_v,
                   g0, g1, w0, w1, isem):
    wid = lax.axis_index("s") * _NC + lax.axis_index("c")
    base = wid * B_PER_W
    ih = [pltpu.async_copy(
        idx_hbm.at[pl.ds(base + j * CHUNK, CHUNK)], idx_v.at[j], isem)
        for j in range(NCHUNK)]
    for h in ih:
        h.wait()
    gsem = (g0, g1)
    wsem = (w0, w1)
    gh = [None, None]
    wh = [None, None]
    gh[0] = pltpu.async_copy(table_hbm.at[idx_v.at[0]], rows_v.at[0], gsem[0])
    for j in range(NCHUNK):
        b = j % 2
        nb = 1 - b
        if j + 1 < NCHUNK:
            if wh[nb] is not None:
                wh[nb].wait()
            gh[nb] = pltpu.async_copy(
                table_hbm.at[idx_v.at[j + 1]], rows_v.at[nb], gsem[nb])
        gh[b].wait()
        wh[b] = pltpu.async_copy(
            rows_v.at[b], out_hbm.at[pl.ds(base + j * CHUNK, CHUNK)], wsem[b])
    for h in wh:
        if h is not None:
            h.wait()


def kernel(rel_ids, clip_embs):
    return _gather_kernel(rel_ids.astype(jnp.int32), clip_embs)
